# baseline (speedup 1.0000x reference)
import jax
import jax.numpy as jnp
from jax.experimental import pallas as pl
from jax.experimental.pallas import tpu as pltpu

_BM = 200


def _gcn_body(x_ref, adj_ref, w_ref, b_ref, out_ref, support_ref):
    @pl.when(pl.program_id(0) == 0)
    def _():
        support_ref[...] = jnp.dot(
            x_ref[...], w_ref[...], preferred_element_type=jnp.float32
        )

    out_ref[...] = (
        jnp.dot(adj_ref[...], support_ref[...], preferred_element_type=jnp.float32)
        + b_ref[...]
    )


def kernel(input, adj, weight, bias):
    n, k = input.shape
    m = adj.shape[0]
    f = weight.shape[1]
    bias2 = bias.reshape(1, f)

    return pl.pallas_call(
        _gcn_body,
        grid=(m // _BM,),
        in_specs=[
            pl.BlockSpec((n, k), lambda i: (0, 0)),
            pl.BlockSpec((_BM, n), lambda i: (i, 0)),
            pl.BlockSpec((k, f), lambda i: (0, 0)),
            pl.BlockSpec((1, f), lambda i: (0, 0)),
        ],
        out_specs=pl.BlockSpec((_BM, f), lambda i: (i, 0)),
        out_shape=jax.ShapeDtypeStruct((m, f), jnp.float32),
        scratch_shapes=[pltpu.VMEM((n, f), jnp.float32)],
    )(input, adj, weight, bias2)


# speedup vs baseline: 1.0033x; 1.0033x over previous
import jax
import jax.numpy as jnp
from jax.experimental import pallas as pl
from jax.experimental.pallas import tpu as pltpu

_BM = 400


def _gcn_body(x_ref, adj_ref, w_ref, b_ref, out_ref, support_ref):
    @pl.when(pl.program_id(0) == 0)
    def _():
        support_ref[...] = jnp.dot(
            x_ref[...], w_ref[...], preferred_element_type=jnp.float32
        )

    out_ref[...] = (
        jnp.dot(adj_ref[...], support_ref[...], preferred_element_type=jnp.float32)
        + b_ref[...]
    )


def kernel(input, adj, weight, bias):
    n, k = input.shape
    m = adj.shape[0]
    f = weight.shape[1]
    bias2 = bias.reshape(1, f)

    return pl.pallas_call(
        _gcn_body,
        grid=(m // _BM,),
        in_specs=[
            pl.BlockSpec((n, k), lambda i: (0, 0)),
            pl.BlockSpec((_BM, n), lambda i: (i, 0)),
            pl.BlockSpec((k, f), lambda i: (0, 0)),
            pl.BlockSpec((1, f), lambda i: (0, 0)),
        ],
        out_specs=pl.BlockSpec((_BM, f), lambda i: (i, 0)),
        out_shape=jax.ShapeDtypeStruct((m, f), jnp.float32),
        scratch_shapes=[pltpu.VMEM((n, f), jnp.float32)],
    )(input, adj, weight, bias2)


# speedup vs baseline: 1.0069x; 1.0036x over previous
import jax
import jax.numpy as jnp
from jax.experimental import pallas as pl
from jax.experimental.pallas import tpu as pltpu

_BM = 400


def _gcn_body(x_ref, adj_ref, w_ref, b_ref, out_ref, support_ref):
    @pl.when(pl.program_id(0) == 0)
    def _():
        support_ref[...] = jnp.dot(
            x_ref[...], w_ref[...], preferred_element_type=jnp.float32
        )

    out_ref[...] = (
        jnp.dot(
            adj_ref[...].astype(jnp.bfloat16),
            support_ref[...].astype(jnp.bfloat16),
            preferred_element_type=jnp.float32,
        )
        + b_ref[...]
    )


def kernel(input, adj, weight, bias):
    n, k = input.shape
    m = adj.shape[0]
    f = weight.shape[1]
    bias2 = bias.reshape(1, f)

    return pl.pallas_call(
        _gcn_body,
        grid=(m // _BM,),
        in_specs=[
            pl.BlockSpec((n, k), lambda i: (0, 0)),
            pl.BlockSpec((_BM, n), lambda i: (i, 0)),
            pl.BlockSpec((k, f), lambda i: (0, 0)),
            pl.BlockSpec((1, f), lambda i: (0, 0)),
        ],
        out_specs=pl.BlockSpec((_BM, f), lambda i: (i, 0)),
        out_shape=jax.ShapeDtypeStruct((m, f), jnp.float32),
        scratch_shapes=[pltpu.VMEM((n, f), jnp.float32)],
    )(input, adj, weight, bias2)
